# Initial kernel scaffold; baseline (speedup 1.0000x reference)
import jax
import jax.numpy as jnp
from jax.experimental import pallas as pl


def kernel(x, pos, batch, mlp1_w, mlp1_b, shortcut_w, shortcut_b, mlp2_w, mlp2_b, lfa1_enc_w, lfa1_enc_b, lfa1_att, lfa1_post_w, lfa1_post_b, lfa2_enc_w, lfa2_enc_b, lfa2_att, lfa2_post_w, lfa2_post_b):
    raise NotImplementedError("write your pallas kernel here")


# speedup vs baseline: 1.0153x; 1.0153x over previous
import functools

import jax
import jax.numpy as jnp
from jax.experimental import pallas as pl
from jax.experimental.pallas import tpu as pltpu

_NEG = 0.2


def _lrelu(v):
    return jnp.where(v > 0, v, _NEG * v)


def _ceil_to(a, m):
    return ((a + m - 1) // m) * m


def _front_body(x_ref, p_ref, wx_ref, bx_ref, wp_ref, bp_ref, xo_ref, po_ref,
                *, d_sc):
    xo = (jnp.dot(x_ref[...], wx_ref[...], preferred_element_type=jnp.float32)
          + bx_ref[...])
    lane = jax.lax.broadcasted_iota(jnp.int32, xo.shape, 1)
    xo_ref[...] = jnp.where((lane < d_sc) | (xo > 0), xo, _NEG * xo)
    po_ref[...] = (jnp.dot(p_ref[...], wp_ref[...],
                           preferred_element_type=jnp.float32) + bp_ref[...])


def _front_call(xp, posp, wx, bx, wp, bp, *, tmf, d_sc):
    npad, d_in = xp.shape
    wxn = wx.shape[1]
    wpn = wp.shape[1]
    return pl.pallas_call(
        functools.partial(_front_body, d_sc=d_sc),
        out_shape=(jax.ShapeDtypeStruct((npad, wxn), jnp.float32),
                   jax.ShapeDtypeStruct((npad, wpn), jnp.float32)),
        grid=(npad // tmf,),
        in_specs=[
            pl.BlockSpec((tmf, d_in), lambda i: (i, 0)),
            pl.BlockSpec((tmf, 3), lambda i: (i, 0)),
            pl.BlockSpec((d_in, wxn), lambda i: (0, 0)),
            pl.BlockSpec((1, wxn), lambda i: (0, 0)),
            pl.BlockSpec((3, wpn), lambda i: (0, 0)),
            pl.BlockSpec((1, wpn), lambda i: (0, 0)),
        ],
        out_specs=(pl.BlockSpec((tmf, wxn), lambda i: (i, 0)),
                   pl.BlockSpec((tmf, wpn), lambda i: (i, 0))),
        compiler_params=pltpu.CompilerParams(
            dimension_semantics=("parallel",),
            vmem_limit_bytes=48 * 1024 * 1024),
    )(xp, posp, wx, bx.reshape(1, wxn), wp, bp.reshape(1, wpn))


def _lfa_body(idx_ref, rin_ref, tbl_ref, wed_ref, watt_ref, wpost_ref, bp_ref,
              *rest, tm, k, ch, tail):
    if tail:
        w2_ref, b2_ref, sc_ref, o_ref = rest
    else:
        (o_ref,) = rest
    f32 = jnp.float32
    te = tm * k
    npad = tbl_ref.shape[0]
    c = 2 * ch

    nbr = idx_ref[...]
    lane = jax.lax.broadcasted_iota(jnp.int32, (te, npad), 1)
    sel = jnp.where(nbr == lane, f32(1.0), f32(0.0))
    g = jnp.dot(sel, tbl_ref[...], preferred_element_type=f32)
    x_j = g[:, :ch]
    pej = g[:, ch:c]
    pos_j = g[:, c:c + 3]

    er = jax.lax.broadcasted_iota(jnp.int32, (te, tm), 0)
    ec = jax.lax.broadcasted_iota(jnp.int32, (te, tm), 1)
    rep = jnp.where(er // k == ec, f32(1.0), f32(0.0))
    ri = jnp.dot(rep, rin_ref[...], preferred_element_type=f32)
    pei = ri[:, :ch]
    pos_i = ri[:, ch:ch + 3]

    d = pos_j - pos_i
    dist = jnp.sqrt(jnp.sum(d * d, axis=1, keepdims=True))
    lse = _lrelu(pej + pei + dist * wed_ref[...])

    att = jnp.dot(jnp.concatenate([x_j, lse], axis=1), watt_ref[...],
                  preferred_element_type=f32)
    att = att - jnp.max(att, axis=0, keepdims=True)
    ex = jnp.exp(att)

    p_in = jnp.concatenate([ex, ex[:, :ch] * x_j, ex[:, ch:] * lse], axis=1)
    pr = jax.lax.broadcasted_iota(jnp.int32, (tm, te), 0)
    pc = jax.lax.broadcasted_iota(jnp.int32, (tm, te), 1)
    pool = jnp.where(pr == pc // k, f32(1.0), f32(0.0))
    agg = jnp.dot(pool, p_in, preferred_element_type=f32)
    den = agg[:, :c]
    aggx = agg[:, c:c + ch] / den[:, :ch]
    aggl = agg[:, c + ch:] / den[:, ch:]

    h = _lrelu(jnp.dot(jnp.concatenate([aggx, aggl], axis=1), wpost_ref[...],
                       preferred_element_type=f32) + bp_ref[...])
    if tail:
        h = _lrelu(jnp.dot(h, w2_ref[...], preferred_element_type=f32)
                   + b2_ref[...] + sc_ref[...])
    o_ref[...] = h


def _lfa_call(idx_col, rin, tbl, wed, watt, wpost, bpost, *, tm, k, ch,
              tail=None):
    npad = tbl.shape[0]
    w_tbl = tbl.shape[1]
    c = 2 * ch
    te = tm * k
    res = lambda i: (0, 0)
    til = lambda i: (i, 0)
    in_specs = [
        pl.BlockSpec((te, 1), til),
        pl.BlockSpec((tm, ch + 3), til),
        pl.BlockSpec((npad, w_tbl), res),
        pl.BlockSpec((1, ch), res),
        pl.BlockSpec((c, c), res),
        pl.BlockSpec((c, c), res),
        pl.BlockSpec((1, c), res),
    ]
    args = [idx_col, rin, tbl, wed.reshape(1, ch), watt, wpost,
            bpost.reshape(1, c)]
    if tail is not None:
        w2, b2, sc = tail
        d_o = w2.shape[1]
        in_specs += [pl.BlockSpec((c, d_o), res),
                     pl.BlockSpec((1, d_o), res),
                     pl.BlockSpec((tm, d_o), til)]
        args += [w2, b2.reshape(1, d_o), sc]
        ow = d_o
    else:
        ow = c
    return pl.pallas_call(
        functools.partial(_lfa_body, tm=tm, k=k, ch=ch, tail=tail is not None),
        out_shape=jax.ShapeDtypeStruct((npad, ow), jnp.float32),
        grid=(npad // tm,),
        in_specs=in_specs,
        out_specs=pl.BlockSpec((tm, ow), til),
        compiler_params=pltpu.CompilerParams(
            dimension_semantics=("parallel",),
            vmem_limit_bytes=64 * 1024 * 1024),
    )(*args)


def kernel(x, pos, batch, mlp1_w, mlp1_b, shortcut_w, shortcut_b, mlp2_w,
           mlp2_b, lfa1_enc_w, lfa1_enc_b, lfa1_att, lfa1_post_w, lfa1_post_b,
           lfa2_enc_w, lfa2_enc_b, lfa2_att, lfa2_post_w, lfa2_post_b):
    kk = 16
    n = x.shape[0]
    ch1 = mlp1_w.shape[1]
    ch2 = 2 * ch1
    d_sc = shortcut_w.shape[1]

    d2 = jnp.sum((pos[:, None, :] - pos[None, :, :]) ** 2, axis=-1)
    same = batch[:, None] == batch[None, :]
    d2 = jnp.where(same, d2, jnp.float32(1e10))
    _, idx = jax.lax.top_k(-d2, kk)

    tm = 128
    while tm > 8 and _ceil_to(n, tm) // tm < 2:
        tm //= 2
    npad = _ceil_to(n, tm)
    pad = npad - n
    xp = jnp.pad(x, ((0, pad), (0, 0)))
    posp = jnp.pad(pos, ((0, pad), (0, 0)))
    idxp = jnp.pad(idx, ((0, pad), (0, 0)))
    idx_col = idxp.reshape(npad * kk, 1).astype(jnp.int32)

    tmf = min(512, npad)
    while npad % tmf:
        tmf //= 2
    if npad // tmf < 2 and tmf >= 16:
        tmf //= 2
        while npad % tmf:
            tmf //= 2

    wei1 = lfa1_enc_w[0:3] - lfa1_enc_w[6:9]
    wej1 = lfa1_enc_w[3:6] + lfa1_enc_w[6:9]
    wed1 = lfa1_enc_w[9:10]
    wei2 = lfa2_enc_w[0:3] - lfa2_enc_w[6:9]
    wej2 = lfa2_enc_w[3:6] + lfa2_enc_w[6:9]
    wed2 = lfa2_enc_w[9:10]

    wx = jnp.concatenate([shortcut_w, mlp1_w], axis=1)
    bx = jnp.concatenate([shortcut_b, mlp1_b])
    wp = jnp.concatenate([wei1, wej1, wei2, wej2], axis=1)
    bp = jnp.concatenate([jnp.zeros_like(lfa1_enc_b), lfa1_enc_b,
                          jnp.zeros_like(lfa2_enc_b), lfa2_enc_b])

    xo, po = _front_call(xp, posp, wx, bx, wp, bp, tmf=tmf, d_sc=d_sc)
    sc = xo[:, :d_sc]
    h1 = xo[:, d_sc:]
    pei1 = po[:, :ch1]
    pej1 = po[:, ch1:2 * ch1]
    pei2 = po[:, 2 * ch1:2 * ch1 + ch2]
    pej2 = po[:, 2 * ch1 + ch2:]

    tbl1 = jnp.concatenate([h1, pej1, posp], axis=1)
    rin1 = jnp.concatenate([pei1, posp], axis=1)
    h2 = _lfa_call(idx_col, rin1, tbl1, wed1, lfa1_att, lfa1_post_w,
                   lfa1_post_b, tm=tm, k=kk, ch=ch1)

    tbl2 = jnp.concatenate([h2, pej2, posp], axis=1)
    rin2 = jnp.concatenate([pei2, posp], axis=1)
    out = _lfa_call(idx_col, rin2, tbl2, wed2, lfa2_att, lfa2_post_w,
                    lfa2_post_b, tm=tm, k=kk, ch=ch2,
                    tail=(mlp2_w, mlp2_b, sc))
    return out[:n], pos, batch


# speedup vs baseline: 6.1239x; 6.0318x over previous
import functools

import jax
import jax.numpy as jnp
from jax.experimental import pallas as pl
from jax.experimental.pallas import tpu as pltpu

_NEG = 0.2


def _lrelu(v):
    return jnp.where(v > 0, v, _NEG * v)


def _ceil_to(a, m):
    return ((a + m - 1) // m) * m


def _knn_body(pi_ref, bi_ref, cand_ref, idx_ref, dst_ref, pen_ref, acc_ref,
              *, tmn, n, k):
    f32 = jnp.float32
    npad = cand_ref.shape[1]
    acc = None
    for c_ in range(3):
        dc = pi_ref[:, c_:c_ + 1] - cand_ref[c_:c_ + 1, :]
        sq = dc * dc
        acc = sq if acc is None else acc + sq
    lane = jax.lax.broadcasted_iota(jnp.int32, (tmn, npad), 1)
    same = bi_ref[...] == cand_ref[3:4, :]
    acc_ref[...] = acc
    d2 = jnp.where(same, acc, f32(1e10))
    pen_ref[...] = jnp.where(lane < n, d2, f32(3e38))
    idx_cols, dst_cols = [], []
    for _ in range(k):
        cur = pen_ref[...]
        m = jnp.min(cur, axis=1, keepdims=True)
        iv = jnp.min(jnp.where(cur == m, lane, jnp.int32(2 ** 30)),
                     axis=1, keepdims=True)
        es = lane == iv
        pen_ref[...] = jnp.where(es, f32(3e38), cur)
        td = jnp.min(jnp.where(es, acc_ref[...], f32(3e38)),
                     axis=1, keepdims=True)
        idx_cols.append(iv)
        dst_cols.append(jnp.sqrt(jnp.maximum(td, 0.0)))
    idx_ref[...] = jnp.concatenate(idx_cols, axis=1)
    dst_ref[...] = jnp.concatenate(dst_cols, axis=1)


def _knn_call(posp, bi, cand, *, tmn, n, k):
    npad = posp.shape[0]
    return pl.pallas_call(
        functools.partial(_knn_body, tmn=tmn, n=n, k=k),
        out_shape=(jax.ShapeDtypeStruct((npad, k), jnp.int32),
                   jax.ShapeDtypeStruct((npad, k), jnp.float32)),
        grid=(npad // tmn,),
        in_specs=[pl.BlockSpec((tmn, 3), lambda i: (i, 0)),
                  pl.BlockSpec((tmn, 1), lambda i: (i, 0)),
                  pl.BlockSpec((8, npad), lambda i: (0, 0))],
        out_specs=(pl.BlockSpec((tmn, k), lambda i: (i, 0)),
                   pl.BlockSpec((tmn, k), lambda i: (i, 0))),
        scratch_shapes=[pltpu.VMEM((tmn, npad), jnp.float32),
                        pltpu.VMEM((tmn, npad), jnp.float32)],
        compiler_params=pltpu.CompilerParams(
            dimension_semantics=("parallel",),
            vmem_limit_bytes=48 * 1024 * 1024),
    )(posp, bi, cand)


def _front_body(x_ref, p_ref, wx_ref, bx_ref, wp_ref, bp_ref, xo_ref, po_ref,
                *, d_sc):
    xo = (jnp.dot(x_ref[...], wx_ref[...], preferred_element_type=jnp.float32)
          + bx_ref[...])
    lane = jax.lax.broadcasted_iota(jnp.int32, xo.shape, 1)
    xo_ref[...] = jnp.where((lane < d_sc) | (xo > 0), xo, _NEG * xo)
    po_ref[...] = (jnp.dot(p_ref[...], wp_ref[...],
                           preferred_element_type=jnp.float32) + bp_ref[...])


def _front_call(xp, posp, wx, bx, wp, bp, *, tmf, d_sc):
    npad, d_in = xp.shape
    wxn = wx.shape[1]
    wpn = wp.shape[1]
    return pl.pallas_call(
        functools.partial(_front_body, d_sc=d_sc),
        out_shape=(jax.ShapeDtypeStruct((npad, wxn), jnp.float32),
                   jax.ShapeDtypeStruct((npad, wpn), jnp.float32)),
        grid=(npad // tmf,),
        in_specs=[
            pl.BlockSpec((tmf, d_in), lambda i: (i, 0)),
            pl.BlockSpec((tmf, 3), lambda i: (i, 0)),
            pl.BlockSpec((d_in, wxn), lambda i: (0, 0)),
            pl.BlockSpec((1, wxn), lambda i: (0, 0)),
            pl.BlockSpec((3, wpn), lambda i: (0, 0)),
            pl.BlockSpec((1, wpn), lambda i: (0, 0)),
        ],
        out_specs=(pl.BlockSpec((tmf, wxn), lambda i: (i, 0)),
                   pl.BlockSpec((tmf, wpn), lambda i: (i, 0))),
        compiler_params=pltpu.CompilerParams(
            dimension_semantics=("parallel",),
            vmem_limit_bytes=48 * 1024 * 1024),
    )(xp, posp, wx, bx.reshape(1, wxn), wp, bp.reshape(1, wpn))


def _lfa_body(idx_ref, dist_ref, rin_ref, tbl_ref, wed_ref, watt_ref,
              wpost_ref, bp_ref,
              *rest, tm, k, ch, tail):
    if tail:
        w2_ref, b2_ref, sc_ref, o_ref = rest
    else:
        (o_ref,) = rest
    f32 = jnp.float32
    te = tm * k
    npad = tbl_ref.shape[0]
    c = 2 * ch

    nbr = idx_ref[...]
    lane = jax.lax.broadcasted_iota(jnp.int32, (te, npad), 1)
    sel = jnp.where(nbr == lane, f32(1.0), f32(0.0))
    g = jnp.dot(sel, tbl_ref[...], preferred_element_type=f32)
    x_j = g[:, :ch]
    pej = g[:, ch:c]

    er = jax.lax.broadcasted_iota(jnp.int32, (te, tm), 0)
    ec = jax.lax.broadcasted_iota(jnp.int32, (te, tm), 1)
    rep = jnp.where(er // k == ec, f32(1.0), f32(0.0))
    pei = jnp.dot(rep, rin_ref[...], preferred_element_type=f32)

    lse = _lrelu(pej + pei + dist_ref[...] * wed_ref[...])

    att = jnp.dot(jnp.concatenate([x_j, lse], axis=1), watt_ref[...],
                  preferred_element_type=f32)
    att = att - jnp.max(att, axis=0, keepdims=True)
    ex = jnp.exp(att)

    p_in = jnp.concatenate([ex, ex[:, :ch] * x_j, ex[:, ch:] * lse], axis=1)
    pr = jax.lax.broadcasted_iota(jnp.int32, (tm, te), 0)
    pc = jax.lax.broadcasted_iota(jnp.int32, (tm, te), 1)
    pool = jnp.where(pr == pc // k, f32(1.0), f32(0.0))
    agg = jnp.dot(pool, p_in, preferred_element_type=f32)
    den = agg[:, :c]
    aggx = agg[:, c:c + ch] / den[:, :ch]
    aggl = agg[:, c + ch:] / den[:, ch:]

    h = _lrelu(jnp.dot(jnp.concatenate([aggx, aggl], axis=1), wpost_ref[...],
                       preferred_element_type=f32) + bp_ref[...])
    if tail:
        h = _lrelu(jnp.dot(h, w2_ref[...], preferred_element_type=f32)
                   + b2_ref[...] + sc_ref[...])
    o_ref[...] = h


def _lfa_call(idx_col, dist_col, rin, tbl, wed, watt, wpost, bpost, *, tm, k,
              ch, tail=None):
    npad = tbl.shape[0]
    w_tbl = tbl.shape[1]
    c = 2 * ch
    te = tm * k
    res = lambda i: (0, 0)
    til = lambda i: (i, 0)
    in_specs = [
        pl.BlockSpec((te, 1), til),
        pl.BlockSpec((te, 1), til),
        pl.BlockSpec((tm, ch), til),
        pl.BlockSpec((npad, w_tbl), res),
        pl.BlockSpec((1, ch), res),
        pl.BlockSpec((c, c), res),
        pl.BlockSpec((c, c), res),
        pl.BlockSpec((1, c), res),
    ]
    args = [idx_col, dist_col, rin, tbl, wed.reshape(1, ch), watt, wpost,
            bpost.reshape(1, c)]
    if tail is not None:
        w2, b2, sc = tail
        d_o = w2.shape[1]
        in_specs += [pl.BlockSpec((c, d_o), res),
                     pl.BlockSpec((1, d_o), res),
                     pl.BlockSpec((tm, d_o), til)]
        args += [w2, b2.reshape(1, d_o), sc]
        ow = d_o
    else:
        ow = c
    return pl.pallas_call(
        functools.partial(_lfa_body, tm=tm, k=k, ch=ch, tail=tail is not None),
        out_shape=jax.ShapeDtypeStruct((npad, ow), jnp.float32),
        grid=(npad // tm,),
        in_specs=in_specs,
        out_specs=pl.BlockSpec((tm, ow), til),
        compiler_params=pltpu.CompilerParams(
            dimension_semantics=("parallel",),
            vmem_limit_bytes=64 * 1024 * 1024),
    )(*args)


def kernel(x, pos, batch, mlp1_w, mlp1_b, shortcut_w, shortcut_b, mlp2_w,
           mlp2_b, lfa1_enc_w, lfa1_enc_b, lfa1_att, lfa1_post_w, lfa1_post_b,
           lfa2_enc_w, lfa2_enc_b, lfa2_att, lfa2_post_w, lfa2_post_b):
    kk = 16
    n = x.shape[0]
    ch1 = mlp1_w.shape[1]
    ch2 = 2 * ch1
    d_sc = shortcut_w.shape[1]

    tm = 128
    while tm > 8 and _ceil_to(n, tm) // tm < 2:
        tm //= 2
    npad = _ceil_to(n, tm)
    pad = npad - n
    xp = jnp.pad(x, ((0, pad), (0, 0)))
    posp = jnp.pad(pos, ((0, pad), (0, 0)))

    tmn = min(256, npad)
    while npad % tmn:
        tmn //= 2
    cand = jnp.zeros((8, npad), jnp.float32)
    cand = cand.at[0:3, :n].set(pos.T)
    cand = cand.at[3, :n].set(batch.astype(jnp.float32))
    bi = jnp.pad(batch.astype(jnp.float32), (0, pad)).reshape(npad, 1)
    idxp, distp = _knn_call(posp, bi, cand, tmn=tmn, n=n, k=kk)
    idx_col = idxp.reshape(npad * kk, 1)
    dist_col = distp.reshape(npad * kk, 1)

    tmf = min(512, npad)
    while npad % tmf:
        tmf //= 2
    if npad // tmf < 2 and tmf >= 16:
        tmf //= 2
        while npad % tmf:
            tmf //= 2

    wei1 = lfa1_enc_w[0:3] - lfa1_enc_w[6:9]
    wej1 = lfa1_enc_w[3:6] + lfa1_enc_w[6:9]
    wed1 = lfa1_enc_w[9:10]
    wei2 = lfa2_enc_w[0:3] - lfa2_enc_w[6:9]
    wej2 = lfa2_enc_w[3:6] + lfa2_enc_w[6:9]
    wed2 = lfa2_enc_w[9:10]

    wx = jnp.concatenate([shortcut_w, mlp1_w], axis=1)
    bx = jnp.concatenate([shortcut_b, mlp1_b])
    wp = jnp.concatenate([wei1, wej1, wei2, wej2], axis=1)
    bp = jnp.concatenate([jnp.zeros_like(lfa1_enc_b), lfa1_enc_b,
                          jnp.zeros_like(lfa2_enc_b), lfa2_enc_b])

    xo, po = _front_call(xp, posp, wx, bx, wp, bp, tmf=tmf, d_sc=d_sc)
    sc = xo[:, :d_sc]
    h1 = xo[:, d_sc:]
    pei1 = po[:, :ch1]
    pej1 = po[:, ch1:2 * ch1]
    pei2 = po[:, 2 * ch1:2 * ch1 + ch2]
    pej2 = po[:, 2 * ch1 + ch2:]

    tbl1 = jnp.concatenate([h1, pej1], axis=1)
    h2 = _lfa_call(idx_col, dist_col, pei1, tbl1, wed1, lfa1_att, lfa1_post_w,
                   lfa1_post_b, tm=tm, k=kk, ch=ch1)

    tbl2 = jnp.concatenate([h2, pej2], axis=1)
    out = _lfa_call(idx_col, dist_col, pei2, tbl2, wed2, lfa2_att,
                    lfa2_post_w, lfa2_post_b, tm=tm, k=kk, ch=ch2,
                    tail=(mlp2_w, mlp2_b, sc))
    return out[:n], pos, batch


# speedup vs baseline: 6.2890x; 1.0270x over previous
import functools

import jax
import jax.numpy as jnp
from jax.experimental import pallas as pl
from jax.experimental.pallas import tpu as pltpu

_NEG = 0.2


def _lrelu(v):
    return jnp.where(v > 0, v, _NEG * v)


def _ceil_to(a, m):
    return ((a + m - 1) // m) * m


def _knn_body(pi_ref, bi_ref, cand_ref, idx_ref, dst_ref, pen_ref, acc_ref,
              *, tmn, n, k):
    f32 = jnp.float32
    npad = cand_ref.shape[1]
    acc = None
    for c_ in range(3):
        dc = pi_ref[:, c_:c_ + 1] - cand_ref[c_:c_ + 1, :]
        sq = dc * dc
        acc = sq if acc is None else acc + sq
    lane = jax.lax.broadcasted_iota(jnp.int32, (tmn, npad), 1)
    same = bi_ref[...] == cand_ref[3:4, :]
    acc_ref[...] = acc
    d2 = jnp.where(same, acc, f32(1e10))
    pen_ref[...] = jnp.where(lane < n, d2, f32(3e38))
    for r in range(k):
        cur = pen_ref[...]
        m = jnp.min(cur, axis=1, keepdims=True)
        iv = jnp.min(jnp.where(cur == m, lane, jnp.int32(2 ** 30)),
                     axis=1, keepdims=True)
        es = lane == iv
        pen_ref[...] = jnp.where(es, f32(3e38), cur)
        idx_ref[:, r:r + 1] = iv
        dst_ref[:, r:r + 1] = jnp.sqrt(jnp.maximum(m, 0.0))

        @pl.when(jnp.any(m >= f32(1e9)))
        def _fix(es=es, m=m, r=r):
            td = jnp.min(jnp.where(es, acc_ref[...], f32(3e38)),
                         axis=1, keepdims=True)
            dst_ref[:, r:r + 1] = jnp.sqrt(jnp.maximum(td, 0.0))


def _knn_call(posp, bi, cand, *, tmn, n, k):
    npad = posp.shape[0]
    return pl.pallas_call(
        functools.partial(_knn_body, tmn=tmn, n=n, k=k),
        out_shape=(jax.ShapeDtypeStruct((npad, k), jnp.int32),
                   jax.ShapeDtypeStruct((npad, k), jnp.float32)),
        grid=(npad // tmn,),
        in_specs=[pl.BlockSpec((tmn, 3), lambda i: (i, 0)),
                  pl.BlockSpec((tmn, 1), lambda i: (i, 0)),
                  pl.BlockSpec((8, npad), lambda i: (0, 0))],
        out_specs=(pl.BlockSpec((tmn, k), lambda i: (i, 0)),
                   pl.BlockSpec((tmn, k), lambda i: (i, 0))),
        scratch_shapes=[pltpu.VMEM((tmn, npad), jnp.float32),
                        pltpu.VMEM((tmn, npad), jnp.float32)],
        compiler_params=pltpu.CompilerParams(
            dimension_semantics=("parallel",),
            vmem_limit_bytes=48 * 1024 * 1024),
    )(posp, bi, cand)


def _front_body(x_ref, p_ref, wx_ref, bx_ref, wp_ref, bp_ref,
                sc_ref, tbl1_ref, pei1_ref, pei2_ref, pej2_ref,
                *, d_sc, ch1, ch2):
    f32 = jnp.float32
    xo = (jnp.dot(x_ref[...], wx_ref[...], preferred_element_type=f32)
          + bx_ref[...])
    po = (jnp.dot(p_ref[...], wp_ref[...], preferred_element_type=f32)
          + bp_ref[...])
    sc_ref[...] = xo[:, :d_sc]
    h1 = _lrelu(xo[:, d_sc:])
    tbl1_ref[...] = jnp.concatenate([h1, po[:, ch1:2 * ch1]], axis=1)
    pei1_ref[...] = po[:, :ch1]
    pei2_ref[...] = po[:, 2 * ch1:2 * ch1 + ch2]
    pej2_ref[...] = po[:, 2 * ch1 + ch2:]


def _front_call(xp, posp, wx, bx, wp, bp, *, tmf, d_sc, ch1, ch2):
    npad, d_in = xp.shape
    wxn = wx.shape[1]
    wpn = wp.shape[1]
    til = lambda i: (i, 0)
    res = lambda i: (0, 0)
    return pl.pallas_call(
        functools.partial(_front_body, d_sc=d_sc, ch1=ch1, ch2=ch2),
        out_shape=(jax.ShapeDtypeStruct((npad, d_sc), jnp.float32),
                   jax.ShapeDtypeStruct((npad, 2 * ch1), jnp.float32),
                   jax.ShapeDtypeStruct((npad, ch1), jnp.float32),
                   jax.ShapeDtypeStruct((npad, ch2), jnp.float32),
                   jax.ShapeDtypeStruct((npad, ch2), jnp.float32)),
        grid=(npad // tmf,),
        in_specs=[
            pl.BlockSpec((tmf, d_in), til),
            pl.BlockSpec((tmf, 3), til),
            pl.BlockSpec((d_in, wxn), res),
            pl.BlockSpec((1, wxn), res),
            pl.BlockSpec((3, wpn), res),
            pl.BlockSpec((1, wpn), res),
        ],
        out_specs=(pl.BlockSpec((tmf, d_sc), til),
                   pl.BlockSpec((tmf, 2 * ch1), til),
                   pl.BlockSpec((tmf, ch1), til),
                   pl.BlockSpec((tmf, ch2), til),
                   pl.BlockSpec((tmf, ch2), til)),
        compiler_params=pltpu.CompilerParams(
            dimension_semantics=("parallel",),
            vmem_limit_bytes=48 * 1024 * 1024),
    )(xp, posp, wx, bx.reshape(1, wxn), wp, bp.reshape(1, wpn))


def _lfa_body(idx_ref, dst_ref, pei_ref, tbl_ref, wed_ref, watt_ref,
              wpost_ref, bp_ref, *rest, k, ch, tail, carry):
    rest = list(rest)
    w2_ref = b2_ref = sc_ref = car_ref = None
    if tail:
        w2_ref, b2_ref, sc_ref = rest[:3]
        rest = rest[3:]
    if carry:
        car_ref = rest[0]
        rest = rest[1:]
    (o_ref,) = rest
    f32 = jnp.float32
    tm = idx_ref.shape[0]
    npad = tbl_ref.shape[0]
    c = 2 * ch

    lane = jax.lax.broadcasted_iota(jnp.int32, (tm, npad), 1)
    pei = pei_ref[...]
    wed = wed_ref[...]
    tbl = tbl_ref[...]
    watt = watt_ref[...]
    xs, ls, atts = [], [], []
    for j in range(k):
        selj = jnp.where(idx_ref[:, j:j + 1] == lane, f32(1.0), f32(0.0))
        gj = jnp.dot(selj, tbl, preferred_element_type=f32)
        xj = gj[:, :ch]
        lsej = _lrelu(gj[:, ch:] + pei + dst_ref[:, j:j + 1] * wed)
        aj = jnp.dot(jnp.concatenate([xj, lsej], axis=1), watt,
                     preferred_element_type=f32)
        xs.append(xj)
        ls.append(lsej)
        atts.append(aj)

    mx = atts[0]
    for j in range(1, k):
        mx = jnp.maximum(mx, atts[j])
    mx = jnp.max(mx, axis=0, keepdims=True)
    den = nx = nl = None
    for j in range(k):
        e = jnp.exp(atts[j] - mx)
        den = e if den is None else den + e
        nxj = e[:, :ch] * xs[j]
        nlj = e[:, ch:] * ls[j]
        nx = nxj if nx is None else nx + nxj
        nl = nlj if nl is None else nl + nlj
    aggx = nx / den[:, :ch]
    aggl = nl / den[:, ch:]

    h = _lrelu(jnp.dot(jnp.concatenate([aggx, aggl], axis=1), wpost_ref[...],
                       preferred_element_type=f32) + bp_ref[...])
    if tail:
        h = _lrelu(jnp.dot(h, w2_ref[...], preferred_element_type=f32)
                   + b2_ref[...] + sc_ref[...])
    if carry:
        o_ref[...] = jnp.concatenate([h, car_ref[...]], axis=1)
    else:
        o_ref[...] = h


def _lfa_call(idxp, distp, pei, tbl, wed, watt, wpost, bpost, *, tm, k, ch,
              tail=None, carry=None):
    npad = tbl.shape[0]
    w_tbl = tbl.shape[1]
    c = 2 * ch
    res = lambda i: (0, 0)
    til = lambda i: (i, 0)
    in_specs = [
        pl.BlockSpec((tm, k), til),
        pl.BlockSpec((tm, k), til),
        pl.BlockSpec((tm, ch), til),
        pl.BlockSpec((npad, w_tbl), res),
        pl.BlockSpec((1, ch), res),
        pl.BlockSpec((c, c), res),
        pl.BlockSpec((c, c), res),
        pl.BlockSpec((1, c), res),
    ]
    args = [idxp, distp, pei, tbl, wed.reshape(1, ch), watt, wpost,
            bpost.reshape(1, c)]
    ow = c
    if tail is not None:
        w2, b2, sc = tail
        d_o = w2.shape[1]
        in_specs += [pl.BlockSpec((c, d_o), res),
                     pl.BlockSpec((1, d_o), res),
                     pl.BlockSpec((tm, d_o), til)]
        args += [w2, b2.reshape(1, d_o), sc]
        ow = d_o
    if carry is not None:
        cw = carry.shape[1]
        in_specs += [pl.BlockSpec((tm, cw), til)]
        args += [carry]
        ow = ow + cw
    return pl.pallas_call(
        functools.partial(_lfa_body, k=k, ch=ch, tail=tail is not None,
                          carry=carry is not None),
        out_shape=jax.ShapeDtypeStruct((npad, ow), jnp.float32),
        grid=(npad // tm,),
        in_specs=in_specs,
        out_specs=pl.BlockSpec((tm, ow), til),
        compiler_params=pltpu.CompilerParams(
            dimension_semantics=("parallel",),
            vmem_limit_bytes=64 * 1024 * 1024),
    )(*args)


def kernel(x, pos, batch, mlp1_w, mlp1_b, shortcut_w, shortcut_b, mlp2_w,
           mlp2_b, lfa1_enc_w, lfa1_enc_b, lfa1_att, lfa1_post_w, lfa1_post_b,
           lfa2_enc_w, lfa2_enc_b, lfa2_att, lfa2_post_w, lfa2_post_b):
    kk = 16
    n = x.shape[0]
    ch1 = mlp1_w.shape[1]
    ch2 = 2 * ch1
    d_sc = shortcut_w.shape[1]

    tm = 128
    while tm > 8 and _ceil_to(n, tm) // tm < 2:
        tm //= 2
    npad = _ceil_to(n, tm)
    pad = npad - n
    xp = jnp.pad(x, ((0, pad), (0, 0)))
    posp = jnp.pad(pos, ((0, pad), (0, 0)))

    tmn = min(256, npad)
    while npad % tmn:
        tmn //= 2
    cand = jnp.zeros((8, npad), jnp.float32)
    cand = cand.at[0:3, :n].set(pos.T)
    cand = cand.at[3, :n].set(batch.astype(jnp.float32))
    bi = jnp.pad(batch.astype(jnp.float32), (0, pad)).reshape(npad, 1)
    idxp, distp = _knn_call(posp, bi, cand, tmn=tmn, n=n, k=kk)

    tmf = min(512, npad)
    while npad % tmf:
        tmf //= 2
    if npad // tmf < 2 and tmf >= 16:
        tmf //= 2
        while npad % tmf:
            tmf //= 2

    wei1 = lfa1_enc_w[0:3] - lfa1_enc_w[6:9]
    wej1 = lfa1_enc_w[3:6] + lfa1_enc_w[6:9]
    wed1 = lfa1_enc_w[9:10]
    wei2 = lfa2_enc_w[0:3] - lfa2_enc_w[6:9]
    wej2 = lfa2_enc_w[3:6] + lfa2_enc_w[6:9]
    wed2 = lfa2_enc_w[9:10]

    wx = jnp.concatenate([shortcut_w, mlp1_w], axis=1)
    bx = jnp.concatenate([shortcut_b, mlp1_b])
    wp = jnp.concatenate([wei1, wej1, wei2, wej2], axis=1)
    bp = jnp.concatenate([jnp.zeros_like(lfa1_enc_b), lfa1_enc_b,
                          jnp.zeros_like(lfa2_enc_b), lfa2_enc_b])

    sc, tbl1, pei1, pei2, pej2 = _front_call(
        xp, posp, wx, bx, wp, bp, tmf=tmf, d_sc=d_sc, ch1=ch1, ch2=ch2)

    tbl2 = _lfa_call(idxp, distp, pei1, tbl1, wed1, lfa1_att, lfa1_post_w,
                     lfa1_post_b, tm=tm, k=kk, ch=ch1, carry=pej2)

    out = _lfa_call(idxp, distp, pei2, tbl2, wed2, lfa2_att, lfa2_post_w,
                    lfa2_post_b, tm=tm, k=kk, ch=ch2,
                    tail=(mlp2_w, mlp2_b, sc))
    return out[:n], pos, batch
